# baseline (speedup 1.0000x reference)
import functools

import jax
import jax.numpy as jnp
from jax import lax
from jax.experimental import pallas as pl
from jax.experimental.pallas import tpu as pltpu
from jax.experimental.pallas import tpu_sc as plsc

_N = 10000
_E = 320000
_D = 128

_NC = 2
_NS = 16
_NW = _NC * _NS
_EPW = _E // _NW
_CH = 100
_CHA = 48
_CHB = _CH - _CHA
_NCH = _EPW // _CH
_NP = 10240
_RPT = _NP // _NS
_CW = 16

_BLK = 1000


def _norm_body(h_ref, o_ref):
    h = h_ref[...]
    n = jnp.sqrt(jnp.sum(h * h, axis=1, keepdims=True)) + 1e-8
    o_ref[...] = h / n


def _normalize(H):
    return pl.pallas_call(
        _norm_body,
        grid=(_N // _BLK,),
        in_specs=[pl.BlockSpec((_BLK, _D), lambda i: (i, 0))],
        out_specs=pl.BlockSpec((_BLK, _D), lambda i: (i, 0)),
        out_shape=jax.ShapeDtypeStruct((_N, _D), jnp.float32),
    )(H)


@functools.cache
def _build_segsum():
    mesh = plsc.VectorSubcoreMesh(core_axis_name="c", subcore_axis_name="s",
                                  num_cores=_NC, num_subcores=_NS)

    @functools.partial(
        pl.kernel,
        out_type=(
            jax.ShapeDtypeStruct((_NC, _NP, _D), jnp.float32),
            jax.ShapeDtypeStruct((_NC, _NP, _CW), jnp.float32),
        ),
        mesh=mesh,
        compiler_params=pltpu.CompilerParams(use_tc_tiling_on_sc=False),
        scratch_types=[
            pltpu.VMEM((_NCH, _CH), jnp.int32),
            pltpu.VMEM((_CH,), jnp.int32),
            pltpu.VMEM((_CH,), jnp.int32),
            pltpu.VMEM((_CH, _D), jnp.float32),
            pltpu.VMEM((_CH, _D), jnp.float32),
            pltpu.VMEM((_CH, _CW), jnp.float32),
            pltpu.VMEM_SHARED((_NP, _D), jnp.float32),
            pltpu.VMEM_SHARED((_NP, _CW), jnp.float32),
            pltpu.SemaphoreType.DMA,
            pltpu.SemaphoreType.DMA,
            pltpu.SemaphoreType.DMA,
            pltpu.SemaphoreType.DMA,
            pltpu.SemaphoreType.DMA,
        ],
    )
    def _segsum(hn, src_r, dst_r, zsum, zcnt, sum_out, cnt_out,
                src_v, dst0_v, dst1_v, rows0_v, rows1_v, ones_v,
                acc_sh, cnt_sh, s0a, s0b, s1a, s1b, osem):
        cid = lax.axis_index("c")
        sid = lax.axis_index("s")
        wid = cid * _NS + sid

        def gather(dst_b, rows_b, sa, sb):
            pltpu.async_copy(hn.at[dst_b.at[pl.ds(0, _CHA)]],
                             rows_b.at[pl.ds(0, _CHA)], sa)
            pltpu.async_copy(hn.at[dst_b.at[pl.ds(_CHA, _CHB)]],
                             rows_b.at[pl.ds(_CHA, _CHB)], sb)

        def gwait(dst_b, rows_b, sa, sb):
            pltpu.make_async_copy(hn.at[dst_b.at[pl.ds(0, _CHA)]],
                                  rows_b.at[pl.ds(0, _CHA)], sa).wait()
            pltpu.make_async_copy(hn.at[dst_b.at[pl.ds(_CHA, _CHB)]],
                                  rows_b.at[pl.ds(_CHA, _CHB)], sb).wait()

        for r in range(_CH):
            ones_v[r, :] = jnp.ones((_CW,), jnp.float32)

        z0 = pl.multiple_of(sid * _RPT, 8)
        pltpu.sync_copy(zsum.at[pl.ds(z0, _RPT)], acc_sh.at[pl.ds(z0, _RPT)])
        pltpu.sync_copy(zcnt.at[pl.ds(z0, _RPT)], cnt_sh.at[pl.ds(z0, _RPT)])

        pltpu.sync_copy(src_r.at[wid], src_v)
        pltpu.sync_copy(dst_r.at[wid, 0], dst0_v)
        plsc.subcore_barrier()

        gather(dst0_v, rows0_v, s0a, s0b)

        def step(j2, carry):
            j = 2 * j2
            pltpu.sync_copy(dst_r.at[wid, j + 1], dst1_v)
            gather(dst1_v, rows1_v, s1a, s1b)
            gwait(dst0_v, rows0_v, s0a, s0b)
            pltpu.async_copy(ones_v, cnt_sh.at[src_v.at[j]], osem, add=True)
            pltpu.sync_copy(rows0_v, acc_sh.at[src_v.at[j]], add=True)

            @pl.when(j2 < _NCH // 2 - 1)
            def _():
                pltpu.sync_copy(dst_r.at[wid, j + 2], dst0_v)
                gather(dst0_v, rows0_v, s0a, s0b)

            gwait(dst1_v, rows1_v, s1a, s1b)
            pltpu.async_copy(ones_v, cnt_sh.at[src_v.at[j + 1]], osem,
                             add=True)
            pltpu.sync_copy(rows1_v, acc_sh.at[src_v.at[j + 1]], add=True)
            return carry

        lax.fori_loop(0, _NCH // 2, step, 0)

        def drain(j, carry):
            pltpu.make_async_copy(ones_v, cnt_sh.at[src_v.at[0]],
                                  osem).wait()
            return carry

        lax.fori_loop(0, _NCH, drain, 0)
        plsc.subcore_barrier()

        for b in range(_RPT // 80):
            r0 = pl.multiple_of(sid * _RPT + b * 80, 8)
            pltpu.sync_copy(acc_sh.at[pl.ds(r0, 80)], rows0_v.at[pl.ds(0, 80)])
            pltpu.sync_copy(rows0_v.at[pl.ds(0, 80)],
                            sum_out.at[cid, pl.ds(r0, 80)])
            pltpu.sync_copy(cnt_sh.at[pl.ds(r0, 80)], ones_v.at[pl.ds(0, 80)])
            pltpu.sync_copy(ones_v.at[pl.ds(0, 80)],
                            cnt_out.at[cid, pl.ds(r0, 80)])

    return _segsum


def _fin_body(a_ref, hn_ref, s_ref, c_ref, o_ref):
    hn = hn_ref[...]
    s = s_ref[0] + s_ref[1]
    cnt = c_ref[0, :, 0:1] + c_ref[1, :, 0:1]
    curv = jnp.clip(a_ref[0], 1e-8, 1.0)
    cos = jnp.sum(hn * s, axis=1, keepdims=True)
    v = (s - cos * hn) * (curv / jnp.maximum(cnt, 1.0))
    n2 = jnp.sqrt(jnp.sum(v * v, axis=1, keepdims=True)) + 1e-8
    o_ref[...] = v / n2


def _finalize(a, hn, sums, cnts):
    return pl.pallas_call(
        _fin_body,
        grid=(_N // _BLK,),
        in_specs=[
            pl.BlockSpec(memory_space=pltpu.SMEM),
            pl.BlockSpec((_BLK, _D), lambda i: (i, 0)),
            pl.BlockSpec((_NC, _BLK, _D), lambda i: (0, i, 0)),
            pl.BlockSpec((_NC, _BLK, _CW), lambda i: (0, i, 0)),
        ],
        out_specs=pl.BlockSpec((_BLK, _D), lambda i: (i, 0)),
        out_shape=jax.ShapeDtypeStruct((_N, _D), jnp.float32),
    )(a, hn, sums, cnts)


@jax.jit
def kernel(t, H, edge_index, a):
    src = edge_index[0].astype(jnp.int32).reshape(_NW, _NCH, _CH)
    dst = edge_index[1].astype(jnp.int32).reshape(_NW, _NCH, _CH)
    hn = _normalize(H)
    zsum = jnp.zeros((_NP, _D), jnp.float32)
    zcnt = jnp.zeros((_NP, _CW), jnp.float32)
    sums, cnts = _build_segsum()(hn, src, dst, zsum, zcnt)
    return _finalize(jnp.reshape(a, (1,)), hn, sums, cnts)


# speedup vs baseline: 1.0286x; 1.0286x over previous
import functools

import jax
import jax.numpy as jnp
from jax import lax
from jax.experimental import pallas as pl
from jax.experimental.pallas import tpu as pltpu
from jax.experimental.pallas import tpu_sc as plsc

_N = 10000
_E = 320000
_D = 128

_NC = 2
_NS = 16
_NW = _NC * _NS
_EPW = _E // _NW
_CH = 100
_CHA = 48
_CHB = _CH - _CHA
_NCH = _EPW // _CH
_NP = 10240
_RPT = _NP // _NS
_CW = 16

_BLK = 1000


def _norm_body(h_ref, o_ref):
    h = h_ref[...]
    n = jnp.sqrt(jnp.sum(h * h, axis=1, keepdims=True)) + 1e-8
    o_ref[...] = h / n


def _normalize(H):
    return pl.pallas_call(
        _norm_body,
        grid=(_N // _BLK,),
        in_specs=[pl.BlockSpec((_BLK, _D), lambda i: (i, 0))],
        out_specs=pl.BlockSpec((_BLK, _D), lambda i: (i, 0)),
        out_shape=jax.ShapeDtypeStruct((_N, _D), jnp.float32),
    )(H)


@functools.cache
def _build_segsum():
    mesh = plsc.VectorSubcoreMesh(core_axis_name="c", subcore_axis_name="s",
                                  num_cores=_NC, num_subcores=_NS)

    @functools.partial(
        pl.kernel,
        out_type=(
            jax.ShapeDtypeStruct((_NC, _NP, _D), jnp.float32),
            jax.ShapeDtypeStruct((_NC, _NP, _CW), jnp.float32),
        ),
        mesh=mesh,
        compiler_params=pltpu.CompilerParams(use_tc_tiling_on_sc=False),
        scratch_types=[
            pltpu.VMEM((_NCH, _CH), jnp.int32),
            pltpu.VMEM((_CH,), jnp.int32),
            pltpu.VMEM((_CH,), jnp.int32),
            pltpu.VMEM((_CH, _D), jnp.float32),
            pltpu.VMEM((_CH, _D), jnp.float32),
            pltpu.VMEM((_CH, _CW), jnp.float32),
            pltpu.VMEM_SHARED((_NP, _D), jnp.float32),
            pltpu.VMEM_SHARED((_NP, _CW), jnp.float32),
            pltpu.SemaphoreType.DMA,
            pltpu.SemaphoreType.DMA,
            pltpu.SemaphoreType.DMA,
            pltpu.SemaphoreType.DMA,
            pltpu.SemaphoreType.DMA,
        ],
    )
    def _segsum(hn, src_r, dst_r, sum_out, cnt_out,
                src_v, dst0_v, dst1_v, rows0_v, rows1_v, ones_v,
                acc_sh, cnt_sh, s0a, s0b, s1a, s1b, osem):
        cid = lax.axis_index("c")
        sid = lax.axis_index("s")
        wid = cid * _NS + sid

        def gather(dst_b, rows_b, sa, sb):
            pltpu.async_copy(hn.at[dst_b.at[pl.ds(0, _CHA)]],
                             rows_b.at[pl.ds(0, _CHA)], sa)
            pltpu.async_copy(hn.at[dst_b.at[pl.ds(_CHA, _CHB)]],
                             rows_b.at[pl.ds(_CHA, _CHB)], sb)

        def gwait(dst_b, rows_b, sa, sb):
            pltpu.make_async_copy(hn.at[dst_b.at[pl.ds(0, _CHA)]],
                                  rows_b.at[pl.ds(0, _CHA)], sa).wait()
            pltpu.make_async_copy(hn.at[dst_b.at[pl.ds(_CHA, _CHB)]],
                                  rows_b.at[pl.ds(_CHA, _CHB)], sb).wait()

        def zrow(r, carry):
            for c in range(_D // 16):
                rows0_v[r, pl.ds(c * 16, 16)] = jnp.zeros((16,), jnp.float32)
            ones_v[r, pl.ds(0, _CW)] = jnp.zeros((_CW,), jnp.float32)
            return carry

        lax.fori_loop(0, _CH, zrow, 0)
        z0 = pl.multiple_of(sid * _RPT, 8)
        for b in range(_RPT // 80):
            zb = pl.multiple_of(z0 + b * 80, 8)
            pltpu.sync_copy(rows0_v.at[pl.ds(0, 80)], acc_sh.at[pl.ds(zb, 80)])
            pltpu.sync_copy(ones_v.at[pl.ds(0, 80)], cnt_sh.at[pl.ds(zb, 80)])

        def orow(r, carry):
            ones_v[r, pl.ds(0, _CW)] = jnp.ones((_CW,), jnp.float32)
            return carry

        lax.fori_loop(0, _CH, orow, 0)

        pltpu.sync_copy(src_r.at[wid], src_v)
        pltpu.sync_copy(dst_r.at[wid, 0], dst0_v)
        plsc.subcore_barrier()

        gather(dst0_v, rows0_v, s0a, s0b)

        def step(j2, carry):
            j = 2 * j2
            pltpu.sync_copy(dst_r.at[wid, j + 1], dst1_v)
            gather(dst1_v, rows1_v, s1a, s1b)
            gwait(dst0_v, rows0_v, s0a, s0b)
            pltpu.async_copy(ones_v, cnt_sh.at[src_v.at[j]], osem, add=True)
            pltpu.sync_copy(rows0_v, acc_sh.at[src_v.at[j]], add=True)

            @pl.when(j2 < _NCH // 2 - 1)
            def _():
                pltpu.sync_copy(dst_r.at[wid, j + 2], dst0_v)
                gather(dst0_v, rows0_v, s0a, s0b)

            gwait(dst1_v, rows1_v, s1a, s1b)
            pltpu.async_copy(ones_v, cnt_sh.at[src_v.at[j + 1]], osem,
                             add=True)
            pltpu.sync_copy(rows1_v, acc_sh.at[src_v.at[j + 1]], add=True)
            return carry

        lax.fori_loop(0, _NCH // 2, step, 0)

        def drain(j, carry):
            pltpu.make_async_copy(ones_v, cnt_sh.at[src_v.at[0]],
                                  osem).wait()
            return carry

        lax.fori_loop(0, _NCH, drain, 0)
        plsc.subcore_barrier()

        for b in range(_RPT // 80):
            r0 = pl.multiple_of(sid * _RPT + b * 80, 8)
            pltpu.sync_copy(acc_sh.at[pl.ds(r0, 80)], rows0_v.at[pl.ds(0, 80)])
            pltpu.sync_copy(rows0_v.at[pl.ds(0, 80)],
                            sum_out.at[cid, pl.ds(r0, 80)])
            pltpu.sync_copy(cnt_sh.at[pl.ds(r0, 80)], ones_v.at[pl.ds(0, 80)])
            pltpu.sync_copy(ones_v.at[pl.ds(0, 80)],
                            cnt_out.at[cid, pl.ds(r0, 80)])

    return _segsum


def _fin_body(a_ref, hn_ref, s_ref, c_ref, o_ref):
    hn = hn_ref[...]
    s = s_ref[0] + s_ref[1]
    cnt = c_ref[0, :, 0:1] + c_ref[1, :, 0:1]
    curv = jnp.clip(a_ref[0], 1e-8, 1.0)
    cos = jnp.sum(hn * s, axis=1, keepdims=True)
    v = (s - cos * hn) * (curv / jnp.maximum(cnt, 1.0))
    n2 = jnp.sqrt(jnp.sum(v * v, axis=1, keepdims=True)) + 1e-8
    o_ref[...] = v / n2


def _finalize(a, hn, sums, cnts):
    return pl.pallas_call(
        _fin_body,
        grid=(_N // _BLK,),
        in_specs=[
            pl.BlockSpec(memory_space=pltpu.SMEM),
            pl.BlockSpec((_BLK, _D), lambda i: (i, 0)),
            pl.BlockSpec((_NC, _BLK, _D), lambda i: (0, i, 0)),
            pl.BlockSpec((_NC, _BLK, _CW), lambda i: (0, i, 0)),
        ],
        out_specs=pl.BlockSpec((_BLK, _D), lambda i: (i, 0)),
        out_shape=jax.ShapeDtypeStruct((_N, _D), jnp.float32),
    )(a, hn, sums, cnts)


@jax.jit
def kernel(t, H, edge_index, a):
    src = edge_index[0].astype(jnp.int32).reshape(_NW, _NCH, _CH)
    dst = edge_index[1].astype(jnp.int32).reshape(_NW, _NCH, _CH)
    hn = _normalize(H)
    sums, cnts = _build_segsum()(hn, src, dst)
    return _finalize(jnp.reshape(a, (1,)), hn, sums, cnts)
